# Initial kernel scaffold; baseline (speedup 1.0000x reference)
import functools

import jax
import jax.numpy as jnp
from jax import lax
from jax.experimental import pallas as pl
from jax.experimental.pallas import tpu as pltpu
from jax.experimental.pallas import tpu_sc as plsc

N = 10000
D = 128
DP = 144
E = 320000
NC = 2
NS = 16
NW = NC * NS
EPW = E // NW
K = 80
NCHUNK = EPW // K
RPT = N // NS


@functools.partial(
    pl.kernel,
    mesh=plsc.VectorSubcoreMesh(core_axis_name="c", subcore_axis_name="s"),
    out_type=jax.ShapeDtypeStruct((NC, N, DP), jnp.float32),
    scratch_types=[
        pltpu.VMEM_SHARED((N, DP), jnp.float32),
        pltpu.VMEM((K,), jnp.int32),
        pltpu.VMEM((K,), jnp.int32),
        pltpu.VMEM((K, DP), jnp.float32),
        pltpu.SemaphoreType.DMA,
    ],
)
def _sc_aggregate(tab, src, dst, zz, out, acc, idx_s, idx_d, rows, sem):
    cid = lax.axis_index("c")
    sid = lax.axis_index("s")
    wid = sid * NC + cid

    pltpu.sync_copy(zz, acc.at[pl.ds(sid * RPT, RPT)])
    plsc.subcore_barrier()

    base = wid * EPW

    def chunk(c, carry):
        off = pl.multiple_of(base + c * K, 8)
        pltpu.sync_copy(src.at[pl.ds(off, K)], idx_s)
        pltpu.async_copy(tab.at[idx_s], rows, sem).wait()
        pltpu.sync_copy(dst.at[pl.ds(off, K)], idx_d)
        pltpu.sync_copy(rows, acc.at[idx_d], add=True)
        return carry

    lax.fori_loop(0, NCHUNK, chunk, 0)
    plsc.subcore_barrier()

    pltpu.sync_copy(acc.at[pl.ds(sid * RPT, RPT)],
                    out.at[cid, pl.ds(sid * RPT, RPT)])


R = 1000


def _affine_body(p_ref, w_ref, b_ref, o_ref, *, relu, pad_out):
    p = p_ref[0] + p_ref[1]
    deg = jnp.maximum(jnp.sum(p[:, D:DP], axis=1, keepdims=True), 1.0)
    a = p[:, :D] / deg
    h = jnp.dot(a, w_ref[...], preferred_element_type=jnp.float32) + b_ref[...]
    if relu:
        h = jnp.maximum(h, 0.0)
    if pad_out:
        lane = lax.broadcasted_iota(jnp.int32, (R, DP - D), 1)
        pad = jnp.where(lane == 0, 1.0, 0.0).astype(jnp.float32)
        o_ref[...] = jnp.concatenate([h, pad], axis=1)
    else:
        o_ref[...] = h


def _tc_affine(partials, w, b, *, relu, pad_out):
    dout = DP if pad_out else D
    return pl.pallas_call(
        functools.partial(_affine_body, relu=relu, pad_out=pad_out),
        grid=(N // R,),
        in_specs=[
            pl.BlockSpec((NC, R, DP), lambda i: (0, i, 0)),
            pl.BlockSpec((D, D), lambda i: (0, 0)),
            pl.BlockSpec((1, D), lambda i: (0, 0)),
        ],
        out_specs=pl.BlockSpec((R, dout), lambda i: (i, 0)),
        out_shape=jax.ShapeDtypeStruct((N, dout), jnp.float32),
    )(partials, w, b.reshape(1, D))


def kernel(x, edge_index_list, W0, b0, W1, b1):
    xp = jnp.concatenate(
        [x, jnp.ones((N, 1), jnp.float32), jnp.zeros((N, DP - D - 1), jnp.float32)],
        axis=1)
    zz = jnp.zeros((RPT, DP), jnp.float32)

    p1 = _sc_aggregate(xp, edge_index_list[0, 0], edge_index_list[0, 1], zz)
    h1 = _tc_affine(p1, W0, b0, relu=True, pad_out=True)
    p2 = _sc_aggregate(h1, edge_index_list[1, 0], edge_index_list[1, 1], zz)
    out = _tc_affine(p2, W1, b1, relu=False, pad_out=False)
    return out


# speedup vs baseline: 4.8231x; 4.8231x over previous
import functools

import jax
import jax.numpy as jnp
from jax import lax
from jax.experimental import pallas as pl
from jax.experimental.pallas import tpu as pltpu
from jax.experimental.pallas import tpu_sc as plsc

N = 10000
D = 128
DP = 144
E = 320000
NC = 2
NS = 16
NW = NC * NS
EPW = E // NW
K = 80
NCHUNK = EPW // K
NP = 10240
RPT = NP // NS


@functools.partial(
    pl.kernel,
    mesh=plsc.VectorSubcoreMesh(core_axis_name="c", subcore_axis_name="s"),
    out_type=jax.ShapeDtypeStruct((NC, NP, DP), jnp.float32),
    scratch_types=[
        pltpu.VMEM_SHARED((NP, DP), jnp.float32),
        pltpu.VMEM((K,), jnp.int32),
        pltpu.VMEM((K,), jnp.int32),
        pltpu.VMEM((K, DP), jnp.float32),
        pltpu.SemaphoreType.DMA,
    ],
    compiler_params=pltpu.CompilerParams(use_tc_tiling_on_sc=False),
)
def _sc_aggregate(tab, src, dst, zz, out, acc, idx_s, idx_d, rows, sem):
    cid = lax.axis_index("c")
    sid = lax.axis_index("s")
    wid = sid * NC + cid

    pltpu.sync_copy(zz, acc.at[pl.ds(sid * RPT, RPT)])
    plsc.subcore_barrier()

    base = wid * EPW

    def chunk(c, carry):
        off = pl.multiple_of(base + c * K, 8)
        pltpu.sync_copy(src.at[pl.ds(off, K)], idx_s)
        pltpu.async_copy(tab.at[idx_s], rows, sem).wait()
        pltpu.sync_copy(dst.at[pl.ds(off, K)], idx_d)
        pltpu.sync_copy(rows, acc.at[idx_d], add=True)
        return carry

    lax.fori_loop(0, NCHUNK, chunk, 0)
    plsc.subcore_barrier()

    pltpu.sync_copy(acc.at[pl.ds(sid * RPT, RPT)],
                    out.at[cid, pl.ds(sid * RPT, RPT)])


R = 1000


def _affine_body(p_ref, w_ref, b_ref, o_ref, *, relu, pad_out):
    p = p_ref[0] + p_ref[1]
    deg = jnp.maximum(jnp.sum(p[:, D:DP], axis=1, keepdims=True), 1.0)
    a = p[:, :D] / deg
    h = jnp.dot(a, w_ref[...], preferred_element_type=jnp.float32) + b_ref[...]
    if relu:
        h = jnp.maximum(h, 0.0)
    if pad_out:
        lane = lax.broadcasted_iota(jnp.int32, (R, DP - D), 1)
        pad = jnp.where(lane == 0, 1.0, 0.0).astype(jnp.float32)
        o_ref[...] = jnp.concatenate([h, pad], axis=1)
    else:
        o_ref[...] = h


def _tc_affine(partials, w, b, *, relu, pad_out):
    dout = DP if pad_out else D
    return pl.pallas_call(
        functools.partial(_affine_body, relu=relu, pad_out=pad_out),
        grid=(N // R,),
        in_specs=[
            pl.BlockSpec((NC, R, DP), lambda i: (0, i, 0)),
            pl.BlockSpec((D, D), lambda i: (0, 0)),
            pl.BlockSpec((1, D), lambda i: (0, 0)),
        ],
        out_specs=pl.BlockSpec((R, dout), lambda i: (i, 0)),
        out_shape=jax.ShapeDtypeStruct((N, dout), jnp.float32),
    )(partials, w, b.reshape(1, D))


def kernel(x, edge_index_list, W0, b0, W1, b1):
    xp = jnp.concatenate(
        [x, jnp.ones((N, 1), jnp.float32), jnp.zeros((N, DP - D - 1), jnp.float32)],
        axis=1)
    zz = jnp.zeros((RPT, DP), jnp.float32)

    p1 = _sc_aggregate(xp, edge_index_list[0, 0], edge_index_list[0, 1], zz)
    h1 = _tc_affine(p1, W0, b0, relu=True, pad_out=True)
    p2 = _sc_aggregate(h1, edge_index_list[1, 0], edge_index_list[1, 1], zz)
    out = _tc_affine(p2, W1, b1, relu=False, pad_out=False)
    return out


# speedup vs baseline: 7.6600x; 1.5882x over previous
import functools

import jax
import jax.numpy as jnp
from jax import lax
from jax.experimental import pallas as pl
from jax.experimental.pallas import tpu as pltpu
from jax.experimental.pallas import tpu_sc as plsc

N = 10000
D = 128
DP = 136
E = 320000
NC = 2
NS = 16
NW = NC * NS
EPW = E // NW
K = 80
NCHUNK = EPW // K
NP = 10240
RPT = NP // NS


@functools.partial(
    pl.kernel,
    mesh=plsc.VectorSubcoreMesh(core_axis_name="c", subcore_axis_name="s"),
    out_type=jax.ShapeDtypeStruct((NC, NP, DP), jnp.float32),
    scratch_types=[
        pltpu.VMEM_SHARED((NP, DP), jnp.float32),
        pltpu.VMEM((NCHUNK, K), jnp.int32),
        pltpu.VMEM((NCHUNK, K), jnp.int32),
        pltpu.VMEM((K, DP), jnp.float32),
        pltpu.VMEM((K, DP), jnp.float32),
        pltpu.SemaphoreType.DMA,
        pltpu.SemaphoreType.DMA,
        pltpu.SemaphoreType.DMA,
        pltpu.SemaphoreType.DMA,
    ],
    compiler_params=pltpu.CompilerParams(use_tc_tiling_on_sc=False),
)
def _sc_aggregate(tab, src, dst, zz, out, acc, idx_s, idx_d, rows_a, rows_b,
                  sem_ga, sem_gb, sem_sa, sem_sb):
    cid = lax.axis_index("c")
    sid = lax.axis_index("s")
    wid = sid * NC + cid

    def gath(c, buf, sem):
        return pltpu.make_async_copy(tab.at[idx_s.at[c]], buf, sem)

    def scat(c, buf, sem):
        return pltpu.make_async_copy(buf, acc.at[idx_d.at[c]], sem)

    pltpu.sync_copy(zz, acc.at[pl.ds(sid * RPT, RPT)])
    pltpu.sync_copy(src.at[wid], idx_s)
    pltpu.sync_copy(dst.at[wid], idx_d)
    gath(0, rows_a, sem_ga).start()
    gath(1, rows_b, sem_gb).start()
    plsc.subcore_barrier()

    gath(0, rows_a, sem_ga).wait()
    scat(0, rows_a, sem_sa).start(add=True)

    def pair(g, carry):
        c0 = 2 * g + 1
        c1 = 2 * g + 2
        gath(c0, rows_b, sem_gb).wait()
        scat(c0, rows_b, sem_sb).start(add=True)
        scat(c0 - 1, rows_a, sem_sa).wait()
        gath(c1, rows_a, sem_ga).start()
        gath(c1, rows_a, sem_ga).wait()
        scat(c1, rows_a, sem_sa).start(add=True)
        scat(c0, rows_b, sem_sb).wait()
        nxt = jnp.minimum(c1 + 1, NCHUNK - 1)
        gath(nxt, rows_b, sem_gb).start()
        return carry

    lax.fori_loop(0, (NCHUNK - 1) // 2, pair, 0)
    scat(NCHUNK - 1, rows_a, sem_sa).wait()
    gath(NCHUNK - 1, rows_b, sem_gb).wait()
    plsc.subcore_barrier()

    pltpu.sync_copy(acc.at[pl.ds(sid * RPT, RPT)],
                    out.at[cid, pl.ds(sid * RPT, RPT)])


R = 1000


def _affine_body(p_ref, w_ref, b_ref, o_ref, *, relu, pad_out):
    p = p_ref[0] + p_ref[1]
    deg = jnp.maximum(jnp.sum(p[:, D:DP], axis=1, keepdims=True), 1.0)
    a = p[:, :D] / deg
    h = jnp.dot(a, w_ref[...], preferred_element_type=jnp.float32) + b_ref[...]
    if relu:
        h = jnp.maximum(h, 0.0)
    if pad_out:
        lane = lax.broadcasted_iota(jnp.int32, (R, DP - D), 1)
        pad = jnp.where(lane == 0, 1.0, 0.0).astype(jnp.float32)
        o_ref[...] = jnp.concatenate([h, pad], axis=1)
    else:
        o_ref[...] = h


def _tc_affine(partials, w, b, *, relu, pad_out):
    dout = DP if pad_out else D
    return pl.pallas_call(
        functools.partial(_affine_body, relu=relu, pad_out=pad_out),
        grid=(N // R,),
        in_specs=[
            pl.BlockSpec((NC, R, DP), lambda i: (0, i, 0)),
            pl.BlockSpec((D, D), lambda i: (0, 0)),
            pl.BlockSpec((1, D), lambda i: (0, 0)),
        ],
        out_specs=pl.BlockSpec((R, dout), lambda i: (i, 0)),
        out_shape=jax.ShapeDtypeStruct((N, dout), jnp.float32),
    )(partials, w, b.reshape(1, D))


def kernel(x, edge_index_list, W0, b0, W1, b1):
    xp = jnp.concatenate(
        [x, jnp.ones((N, 1), jnp.float32), jnp.zeros((N, DP - D - 1), jnp.float32)],
        axis=1)
    zz = jnp.zeros((RPT, DP), jnp.float32)

    e = edge_index_list.reshape(2, 2, NW, NCHUNK, K)
    p1 = _sc_aggregate(xp, e[0, 0], e[0, 1], zz)
    h1 = _tc_affine(p1, W0, b0, relu=True, pad_out=True)
    p2 = _sc_aggregate(h1, e[1, 0], e[1, 1], zz)
    out = _tc_affine(p2, W1, b1, relu=False, pad_out=False)
    return out


# speedup vs baseline: 8.0822x; 1.0551x over previous
import functools

import jax
import jax.numpy as jnp
from jax import lax
from jax.experimental import pallas as pl
from jax.experimental.pallas import tpu as pltpu
from jax.experimental.pallas import tpu_sc as plsc

N = 10000
D = 128
DP = 136
E = 320000
NC = 2
NS = 16
NW = NC * NS
EPW = E // NW
K = 40
NCHUNK = EPW // K
NP = 10240
RPT = NP // NS


@functools.partial(
    pl.kernel,
    mesh=plsc.VectorSubcoreMesh(core_axis_name="c", subcore_axis_name="s"),
    out_type=jax.ShapeDtypeStruct((NC, NP, DP), jnp.float32),
    scratch_types=[
        pltpu.VMEM_SHARED((NP, DP), jnp.float32),
        pltpu.VMEM((NCHUNK, K), jnp.int32),
        pltpu.VMEM((NCHUNK, K), jnp.int32),
        pltpu.VMEM((4, K, DP), jnp.float32),
        pltpu.SemaphoreType.DMA,
        pltpu.SemaphoreType.DMA,
        pltpu.SemaphoreType.DMA,
        pltpu.SemaphoreType.DMA,
        pltpu.SemaphoreType.DMA,
        pltpu.SemaphoreType.DMA,
        pltpu.SemaphoreType.DMA,
        pltpu.SemaphoreType.DMA,
    ],
    compiler_params=pltpu.CompilerParams(use_tc_tiling_on_sc=False),
)
def _sc_aggregate(tab, src, dst, zz, out, acc, idx_s, idx_d, rows,
                  g0, g1, g2, g3, s0, s1, s2, s3):
    cid = lax.axis_index("c")
    sid = lax.axis_index("s")
    wid = sid * NC + cid
    sem_g = (g0, g1, g2, g3)
    sem_s = (s0, s1, s2, s3)

    def gath(c, b):
        return pltpu.make_async_copy(tab.at[idx_s.at[c]], rows.at[b], sem_g[b])

    def scat(c, b):
        return pltpu.make_async_copy(rows.at[b], acc.at[idx_d.at[c]], sem_s[b])

    pltpu.sync_copy(zz, acc.at[pl.ds(sid * RPT, RPT)])
    pltpu.sync_copy(src.at[wid], idx_s)
    pltpu.sync_copy(dst.at[wid], idx_d)
    gath(0, 0).start()
    gath(1, 1).start()
    plsc.subcore_barrier()

    gath(0, 0).wait()
    scat(0, 0).start(add=True)
    gath(2, 2).start()
    gath(1, 1).wait()
    scat(1, 1).start(add=True)
    gath(3, 3).start()

    def group(m, carry):
        for j in range(4):
            c = 2 + 4 * m + j
            b = (2 + j) % 4
            gath(c, b).wait()
            scat(c, b).start(add=True)
            scat(c - 2, (b + 2) % 4).wait()
            nxt = jnp.minimum(c + 2, NCHUNK - 1)
            gath(nxt, (b + 2) % 4).start()
        return carry

    lax.fori_loop(0, (NCHUNK - 2) // 4, group, 0)
    scat(NCHUNK - 2, (NCHUNK - 2) % 4).wait()
    scat(NCHUNK - 1, (NCHUNK - 1) % 4).wait()
    gath(NCHUNK - 1, NCHUNK % 4).wait()
    gath(NCHUNK - 1, (NCHUNK + 1) % 4).wait()
    plsc.subcore_barrier()

    pltpu.sync_copy(acc.at[pl.ds(sid * RPT, RPT)],
                    out.at[cid, pl.ds(sid * RPT, RPT)])


R = 1000


def _affine_body(p_ref, w_ref, b_ref, o_ref, *, relu, pad_out):
    p = p_ref[0] + p_ref[1]
    deg = jnp.maximum(jnp.sum(p[:, D:DP], axis=1, keepdims=True), 1.0)
    a = p[:, :D] / deg
    h = jnp.dot(a, w_ref[...], preferred_element_type=jnp.float32) + b_ref[...]
    if relu:
        h = jnp.maximum(h, 0.0)
    if pad_out:
        lane = lax.broadcasted_iota(jnp.int32, (R, DP - D), 1)
        pad = jnp.where(lane == 0, 1.0, 0.0).astype(jnp.float32)
        o_ref[...] = jnp.concatenate([h, pad], axis=1)
    else:
        o_ref[...] = h


def _tc_affine(partials, w, b, *, relu, pad_out):
    dout = DP if pad_out else D
    return pl.pallas_call(
        functools.partial(_affine_body, relu=relu, pad_out=pad_out),
        grid=(N // R,),
        in_specs=[
            pl.BlockSpec((NC, R, DP), lambda i: (0, i, 0)),
            pl.BlockSpec((D, D), lambda i: (0, 0)),
            pl.BlockSpec((1, D), lambda i: (0, 0)),
        ],
        out_specs=pl.BlockSpec((R, dout), lambda i: (i, 0)),
        out_shape=jax.ShapeDtypeStruct((N, dout), jnp.float32),
    )(partials, w, b.reshape(1, D))


def kernel(x, edge_index_list, W0, b0, W1, b1):
    xp = jnp.concatenate(
        [x, jnp.ones((N, 1), jnp.float32), jnp.zeros((N, DP - D - 1), jnp.float32)],
        axis=1)
    zz = jnp.zeros((RPT, DP), jnp.float32)

    e = edge_index_list.reshape(2, 2, NW, NCHUNK, K)
    p1 = _sc_aggregate(xp, e[0, 0], e[0, 1], zz)
    h1 = _tc_affine(p1, W0, b0, relu=True, pad_out=True)
    p2 = _sc_aggregate(h1, e[1, 0], e[1, 1], zz)
    out = _tc_affine(p2, W1, b1, relu=False, pad_out=False)
    return out


# speedup vs baseline: 9.8850x; 1.2231x over previous
import functools

import jax
import jax.numpy as jnp
from jax import lax
from jax.experimental import pallas as pl
from jax.experimental.pallas import tpu as pltpu
from jax.experimental.pallas import tpu_sc as plsc

N = 10000
D = 128
E = 320000
NC = 2
NS = 16
NW = NC * NS
EPW = E // NW
K = 40
NCHUNK = EPW // K
NP = 10240
RPT = NP // NS


@functools.partial(
    pl.kernel,
    mesh=plsc.VectorSubcoreMesh(core_axis_name="c", subcore_axis_name="s"),
    out_type=(
        jax.ShapeDtypeStruct((NC, NP, D), jnp.float32),
        jax.ShapeDtypeStruct((NW, N), jnp.float32),
    ),
    scratch_types=[
        pltpu.VMEM_SHARED((NP, D), jnp.float32),
        pltpu.VMEM((NCHUNK, K), jnp.int32),
        pltpu.VMEM((NCHUNK, K), jnp.int32),
        pltpu.VMEM((3, K, D), jnp.float32),
        pltpu.VMEM((N,), jnp.float32),
        pltpu.SemaphoreType.DMA,
        pltpu.SemaphoreType.DMA,
        pltpu.SemaphoreType.DMA,
        pltpu.SemaphoreType.DMA,
        pltpu.SemaphoreType.DMA,
        pltpu.SemaphoreType.DMA,
        pltpu.SemaphoreType.DMA,
    ],
    compiler_params=pltpu.CompilerParams(use_tc_tiling_on_sc=False,
                                         needs_layout_passes=False),
)
def _sc_aggregate(tab, src, dst, zz, out, out_deg, acc, idx_s, idx_d, rows,
                  hist, g0, g1, g2, s0, s1, s2, sem_h):
    cid = lax.axis_index("c")
    sid = lax.axis_index("s")
    wid = sid * NC + cid
    sem_g = (g0, g1, g2)
    sem_s = (s0, s1, s2)

    def gath(c, b):
        return pltpu.make_async_copy(tab.at[idx_s.at[c]], rows.at[b], sem_g[b])

    def scat(c, b):
        return pltpu.make_async_copy(rows.at[b], acc.at[idx_d.at[c]], sem_s[b])

    pltpu.sync_copy(zz, acc.at[pl.ds(sid * RPT, RPT)])
    pltpu.sync_copy(src.at[wid], idx_s)
    pltpu.sync_copy(dst.at[wid], idx_d)
    gath(0, 0).start()
    gath(1, 1).start()
    gath(2, 2).start()

    def hzero(i, carry):
        hist[pl.ds(i * 16, 16)] = jnp.zeros((16,), jnp.float32)
        return carry

    lax.fori_loop(0, N // 16, hzero, 0)
    ones = jnp.ones((16,), jnp.float32)
    tailmask = lax.iota(jnp.int32, 16) >= 8

    def hadd(c, carry):
        plsc.addupdate_scatter(hist, [idx_d[c, pl.ds(0, 16)]], ones)
        plsc.addupdate_scatter(hist, [idx_d[c, pl.ds(16, 16)]], ones)
        plsc.addupdate_scatter(hist, [idx_d[c, pl.ds(24, 16)]], ones,
                               mask=tailmask)
        return carry

    lax.fori_loop(0, NCHUNK, hadd, 0)
    deg_out = pltpu.make_async_copy(hist, out_deg.at[wid], sem_h)
    deg_out.start()
    plsc.subcore_barrier()

    gath(0, 0).wait()
    scat(0, 0).start(add=True)

    def group(m, carry):
        for j in range(3):
            c = 1 + 3 * m + j
            b = (1 + j) % 3
            gath(c, b).wait()
            scat(c, b).start(add=True)
            scat(c - 1, (b + 2) % 3).wait()
            nxt = jnp.minimum(c + 2, NCHUNK - 1)
            gath(nxt, (b + 2) % 3).start()
        return carry

    lax.fori_loop(0, (NCHUNK - 1) // 3, group, 0)
    scat(NCHUNK - 1, (NCHUNK - 1) % 3).wait()
    gath(NCHUNK - 1, NCHUNK % 3).wait()
    gath(NCHUNK - 1, (NCHUNK + 1) % 3).wait()
    deg_out.wait()
    plsc.subcore_barrier()

    pltpu.sync_copy(acc.at[pl.ds(sid * RPT, RPT)],
                    out.at[cid, pl.ds(sid * RPT, RPT)])


R = 1000


def _affine_body(p_ref, deg_ref, w_ref, b_ref, o_ref, *, relu):
    p = p_ref[0] + p_ref[1]
    a = p / deg_ref[...]
    h = jnp.dot(a, w_ref[...], preferred_element_type=jnp.float32) + b_ref[...]
    if relu:
        h = jnp.maximum(h, 0.0)
    o_ref[...] = h


def _tc_affine(partials, deg, w, b, *, relu):
    return pl.pallas_call(
        functools.partial(_affine_body, relu=relu),
        grid=(N // R,),
        in_specs=[
            pl.BlockSpec((NC, R, D), lambda i: (0, i, 0)),
            pl.BlockSpec((R, 1), lambda i: (i, 0)),
            pl.BlockSpec((D, D), lambda i: (0, 0)),
            pl.BlockSpec((1, D), lambda i: (0, 0)),
        ],
        out_specs=pl.BlockSpec((R, D), lambda i: (i, 0)),
        out_shape=jax.ShapeDtypeStruct((N, D), jnp.float32),
    )(partials, deg, w, b.reshape(1, D))


def kernel(x, edge_index_list, W0, b0, W1, b1):
    zz = jnp.zeros((RPT, D), jnp.float32)
    e = edge_index_list.reshape(2, 2, NW, NCHUNK, K)

    p1, hist1 = _sc_aggregate(x, e[0, 0], e[0, 1], zz)
    deg1 = jnp.maximum(hist1.sum(axis=0), 1.0)[:, None]
    h1 = _tc_affine(p1, deg1, W0, b0, relu=True)
    p2, hist2 = _sc_aggregate(h1, e[1, 0], e[1, 1], zz)
    deg2 = jnp.maximum(hist2.sum(axis=0), 1.0)[:, None]
    out = _tc_affine(p2, deg2, W1, b1, relu=False)
    return out


# speedup vs baseline: 10.7124x; 1.0837x over previous
import functools

import jax
import jax.numpy as jnp
from jax import lax
from jax.experimental import pallas as pl
from jax.experimental.pallas import tpu as pltpu
from jax.experimental.pallas import tpu_sc as plsc

N = 10000
D = 128
E = 320000
NC = 2
NS = 16
NW = NC * NS
EPW = E // NW
K = 40
NCHUNK = EPW // K
NP = 10240
RPT = NP // NS


@functools.partial(
    pl.kernel,
    mesh=plsc.VectorSubcoreMesh(core_axis_name="c", subcore_axis_name="s"),
    out_type=(
        jax.ShapeDtypeStruct((NC, NP, D), jnp.float32),
        jax.ShapeDtypeStruct((NW, N), jnp.float32),
    ),
    scratch_types=[
        pltpu.VMEM_SHARED((NP, D), jnp.float32),
        pltpu.VMEM((NCHUNK, K), jnp.int32),
        pltpu.VMEM((NCHUNK, K), jnp.int32),
        pltpu.VMEM((3, K, D), jnp.float32),
        pltpu.VMEM((N,), jnp.float32),
        pltpu.SemaphoreType.DMA,
        pltpu.SemaphoreType.DMA,
        pltpu.SemaphoreType.DMA,
        pltpu.SemaphoreType.DMA,
        pltpu.SemaphoreType.DMA,
        pltpu.SemaphoreType.DMA,
        pltpu.SemaphoreType.DMA,
    ],
    compiler_params=pltpu.CompilerParams(use_tc_tiling_on_sc=False,
                                         needs_layout_passes=False),
)
def _sc_aggregate(tab, src, dst, zz, out, out_deg, acc, idx_s, idx_d, rows,
                  hist, g0, g1, g2, s0, s1, s2, sem_h):
    cid = lax.axis_index("c")
    sid = lax.axis_index("s")
    wid = sid * NC + cid
    sem_g = (g0, g1, g2)
    sem_s = (s0, s1, s2)

    def gath(c, b):
        return pltpu.make_async_copy(tab.at[idx_s.at[c]], rows.at[b], sem_g[b])

    def scat(c, b):
        return pltpu.make_async_copy(rows.at[b], acc.at[idx_d.at[c]], sem_s[b])

    pltpu.sync_copy(zz, acc.at[pl.ds(sid * RPT, RPT)])
    pltpu.sync_copy(src.at[wid], idx_s)
    pltpu.sync_copy(dst.at[wid], idx_d)
    gath(0, 0).start()
    gath(1, 1).start()
    gath(2, 2).start()

    def hzero(i, carry):
        hist[pl.ds(i * 16, 16)] = jnp.zeros((16,), jnp.float32)
        return carry

    lax.fori_loop(0, N // 16, hzero, 0)
    ones = jnp.ones((16,), jnp.float32)
    tailmask = lax.iota(jnp.int32, 16) >= 8

    def hadd(c, carry):
        plsc.addupdate_scatter(hist, [idx_d[c, pl.ds(0, 16)]], ones)
        plsc.addupdate_scatter(hist, [idx_d[c, pl.ds(16, 16)]], ones)
        plsc.addupdate_scatter(hist, [idx_d[c, pl.ds(24, 16)]], ones,
                               mask=tailmask)
        return carry

    lax.fori_loop(0, NCHUNK, hadd, 0)
    deg_out = pltpu.make_async_copy(hist, out_deg.at[wid], sem_h)
    deg_out.start()
    plsc.subcore_barrier()

    gath(0, 0).wait()
    scat(0, 0).start(add=True)

    def group(m, carry):
        for j in range(3):
            c = 1 + 3 * m + j
            b = (1 + j) % 3
            scat(c - 1, (b + 2) % 3).wait()
            nxt = jnp.minimum(c + 2, NCHUNK - 1)
            gath(nxt, (b + 2) % 3).start()
            gath(c, b).wait()
            scat(c, b).start(add=True)
        return carry

    lax.fori_loop(0, (NCHUNK - 1) // 3, group, 0)
    scat(NCHUNK - 1, (NCHUNK - 1) % 3).wait()
    gath(NCHUNK - 1, NCHUNK % 3).wait()
    gath(NCHUNK - 1, (NCHUNK + 1) % 3).wait()
    deg_out.wait()
    plsc.subcore_barrier()

    pltpu.sync_copy(acc.at[pl.ds(sid * RPT, RPT)],
                    out.at[cid, pl.ds(sid * RPT, RPT)])


R = 1000


def _affine_body(p_ref, deg_ref, w_ref, b_ref, o_ref, *, relu):
    p = p_ref[0] + p_ref[1]
    a = p / deg_ref[...]
    h = jnp.dot(a, w_ref[...], preferred_element_type=jnp.float32) + b_ref[...]
    if relu:
        h = jnp.maximum(h, 0.0)
    o_ref[...] = h


def _tc_affine(partials, deg, w, b, *, relu):
    return pl.pallas_call(
        functools.partial(_affine_body, relu=relu),
        grid=(N // R,),
        in_specs=[
            pl.BlockSpec((NC, R, D), lambda i: (0, i, 0)),
            pl.BlockSpec((R, 1), lambda i: (i, 0)),
            pl.BlockSpec((D, D), lambda i: (0, 0)),
            pl.BlockSpec((1, D), lambda i: (0, 0)),
        ],
        out_specs=pl.BlockSpec((R, D), lambda i: (i, 0)),
        out_shape=jax.ShapeDtypeStruct((N, D), jnp.float32),
    )(partials, deg, w, b.reshape(1, D))


def kernel(x, edge_index_list, W0, b0, W1, b1):
    zz = jnp.zeros((RPT, D), jnp.float32)
    e = edge_index_list.reshape(2, 2, NW, NCHUNK, K)

    p1, hist1 = _sc_aggregate(x, e[0, 0], e[0, 1], zz)
    deg1 = jnp.maximum(hist1.sum(axis=0), 1.0)[:, None]
    h1 = _tc_affine(p1, deg1, W0, b0, relu=True)
    p2, hist2 = _sc_aggregate(h1, e[1, 0], e[1, 1], zz)
    deg2 = jnp.maximum(hist2.sum(axis=0), 1.0)[:, None]
    out = _tc_affine(p2, deg2, W1, b1, relu=False)
    return out


# speedup vs baseline: 12.0378x; 1.1237x over previous
import functools

import jax
import jax.numpy as jnp
from jax import lax
from jax.experimental import pallas as pl
from jax.experimental.pallas import tpu as pltpu
from jax.experimental.pallas import tpu_sc as plsc

N = 10000
D = 128
E = 320000
NC = 2
NS = 16
NW = NC * NS
EPW = E // NW
K = 40
NCHUNK = EPW // K
NP = 10240
RPT = NP // NS


@functools.partial(
    pl.kernel,
    mesh=plsc.VectorSubcoreMesh(core_axis_name="c", subcore_axis_name="s"),
    out_type=(
        jax.ShapeDtypeStruct((NC, NP, D), jnp.float32),
        jax.ShapeDtypeStruct((NW, NP // D, D), jnp.float32),
    ),
    scratch_types=[
        pltpu.VMEM_SHARED((NP, D), jnp.float32),
        pltpu.VMEM((NCHUNK, K), jnp.int32),
        pltpu.VMEM((NCHUNK, K), jnp.int32),
        pltpu.VMEM((5 * K, D), jnp.float32),
        pltpu.SemaphoreType.DMA,
        pltpu.SemaphoreType.DMA,
        pltpu.SemaphoreType.DMA,
        pltpu.SemaphoreType.DMA,
        pltpu.SemaphoreType.DMA,
        pltpu.SemaphoreType.DMA,
        pltpu.SemaphoreType.DMA,
        pltpu.SemaphoreType.DMA,
        pltpu.SemaphoreType.DMA,
        pltpu.SemaphoreType.DMA,
    ],
    compiler_params=pltpu.CompilerParams(use_tc_tiling_on_sc=False,
                                         needs_layout_passes=False),
)
def _sc_aggregate(tab, src, dst, zz, out, out_deg, acc, idx_s, idx_d, rows,
                  g0, g1, g2, g3, g4, s0, s1, s2, s3, s4):
    cid = lax.axis_index("c")
    sid = lax.axis_index("s")
    wid = sid * NC + cid
    sem_g = (g0, g1, g2, g3, g4)
    sem_s = (s0, s1, s2, s3, s4)
    HR = NP // D

    def gath(c, b):
        return pltpu.make_async_copy(tab.at[idx_s.at[c]],
                                     rows.at[pl.ds(b * K, K)], sem_g[b])

    def scat(c, b):
        return pltpu.make_async_copy(rows.at[pl.ds(b * K, K)],
                                     acc.at[idx_d.at[c]], sem_s[b])

    pltpu.sync_copy(src.at[wid], idx_s)
    pltpu.sync_copy(dst.at[wid], idx_d)

    def hzero(t, carry):
        rows[t >> 3, pl.ds((t & 7) * 16, 16)] = jnp.zeros((16,), jnp.float32)
        return carry

    lax.fori_loop(0, HR * 8, hzero, 0)
    ones = jnp.ones((16,), jnp.float32)
    tailmask = lax.iota(jnp.int32, 16) >= 8

    def hadd1(v, mask=None):
        plsc.addupdate_scatter(rows, [v >> 7, v & 127], ones, mask=mask)

    def hadd(c, carry):
        hadd1(idx_d[c, pl.ds(0, 16)])
        hadd1(idx_d[c, pl.ds(16, 16)])
        hadd1(idx_d[c, pl.ds(24, 16)], tailmask)
        return carry

    lax.fori_loop(0, NCHUNK, hadd, 0)
    pltpu.sync_copy(rows.at[pl.ds(0, HR)], out_deg.at[wid])

    pltpu.sync_copy(zz, acc.at[pl.ds(sid * RPT, RPT)])
    gath(0, 0).start()
    gath(1, 1).start()
    gath(2, 2).start()
    plsc.subcore_barrier()

    gath(0, 0).wait()
    scat(0, 0).start(add=True)
    gath(3, 3).start()
    gath(1, 1).wait()
    scat(1, 1).start(add=True)
    gath(4, 4).start()
    for c in (2, 3, 4):
        scat(c - 2, c - 2).wait()
        gath(c + 3, c - 2).start()
        gath(c, c).wait()
        scat(c, c).start(add=True)

    def group(m, carry):
        for j in range(5):
            c = 5 * m + j
            bn = (j + 3) % 5
            scat(c - 2, bn).wait()
            nxt = jnp.minimum(c + 3, NCHUNK - 1)
            gath(nxt, bn).start()
            gath(c, j).wait()
            scat(c, j).start(add=True)
        return carry

    lax.fori_loop(1, NCHUNK // 5, group, 0)
    scat(NCHUNK - 2, 3).wait()
    scat(NCHUNK - 1, 4).wait()
    gath(NCHUNK - 1, 0).wait()
    gath(NCHUNK - 1, 1).wait()
    gath(NCHUNK - 1, 2).wait()
    plsc.subcore_barrier()

    pltpu.sync_copy(acc.at[pl.ds(sid * RPT, RPT)],
                    out.at[cid, pl.ds(sid * RPT, RPT)])


R = 1000


def _affine_body(p_ref, deg_ref, w_ref, b_ref, o_ref, *, relu):
    p = p_ref[0] + p_ref[1]
    a = p / deg_ref[...]
    h = jnp.dot(a, w_ref[...], preferred_element_type=jnp.float32) + b_ref[...]
    if relu:
        h = jnp.maximum(h, 0.0)
    o_ref[...] = h


def _tc_affine(partials, deg, w, b, *, relu):
    return pl.pallas_call(
        functools.partial(_affine_body, relu=relu),
        grid=(N // R,),
        in_specs=[
            pl.BlockSpec((NC, R, D), lambda i: (0, i, 0)),
            pl.BlockSpec((R, 1), lambda i: (i, 0)),
            pl.BlockSpec((D, D), lambda i: (0, 0)),
            pl.BlockSpec((1, D), lambda i: (0, 0)),
        ],
        out_specs=pl.BlockSpec((R, D), lambda i: (i, 0)),
        out_shape=jax.ShapeDtypeStruct((N, D), jnp.float32),
    )(partials, deg, w, b.reshape(1, D))


def kernel(x, edge_index_list, W0, b0, W1, b1):
    zz = jnp.zeros((RPT, D), jnp.float32)
    e = edge_index_list.reshape(2, 2, NW, NCHUNK, K)

    p1, hist1 = _sc_aggregate(x, e[0, 0], e[0, 1], zz)
    deg1 = jnp.maximum(hist1.sum(axis=0).reshape(NP)[:N], 1.0)[:, None]
    h1 = _tc_affine(p1, deg1, W0, b0, relu=True)
    p2, hist2 = _sc_aggregate(h1, e[1, 0], e[1, 1], zz)
    deg2 = jnp.maximum(hist2.sum(axis=0).reshape(NP)[:N], 1.0)[:, None]
    out = _tc_affine(p2, deg2, W1, b1, relu=False)
    return out


# speedup vs baseline: 12.1895x; 1.0126x over previous
import functools

import jax
import jax.numpy as jnp
from jax import lax
from jax.experimental import pallas as pl
from jax.experimental.pallas import tpu as pltpu
from jax.experimental.pallas import tpu_sc as plsc

N = 10000
D = 128
E = 320000
NC = 2
NS = 16
NW = NC * NS
EPW = E // NW
K = 40
NCHUNK = EPW // K
NP = 10240
RPT = NP // NS


@functools.partial(
    pl.kernel,
    mesh=plsc.VectorSubcoreMesh(core_axis_name="c", subcore_axis_name="s"),
    out_type=(
        jax.ShapeDtypeStruct((NC, NP, D), jnp.float32),
        jax.ShapeDtypeStruct((NW, NP // D, D), jnp.float32),
    ),
    scratch_types=[
        pltpu.VMEM_SHARED((NP, D), jnp.float32),
        pltpu.VMEM((NCHUNK, K), jnp.int32),
        pltpu.VMEM((NCHUNK, K), jnp.int32),
        pltpu.VMEM((5 * K, D), jnp.float32),
        pltpu.SemaphoreType.DMA,
        pltpu.SemaphoreType.DMA,
        pltpu.SemaphoreType.DMA,
        pltpu.SemaphoreType.DMA,
        pltpu.SemaphoreType.DMA,
        pltpu.SemaphoreType.DMA,
        pltpu.SemaphoreType.DMA,
        pltpu.SemaphoreType.DMA,
        pltpu.SemaphoreType.DMA,
        pltpu.SemaphoreType.DMA,
        pltpu.SemaphoreType.DMA,
        pltpu.SemaphoreType.DMA,
    ],
    compiler_params=pltpu.CompilerParams(use_tc_tiling_on_sc=False,
                                         needs_layout_passes=False),
)
def _sc_aggregate(tab, src, dst, zz, out, out_deg, acc, idx_s, idx_d, rows,
                  g0, g1, g2, g3, g4, s0, s1, s2, s3, s4, z0, z1):
    cid = lax.axis_index("c")
    sid = lax.axis_index("s")
    wid = sid * NC + cid
    sem_g = (g0, g1, g2, g3, g4)
    sem_s = (s0, s1, s2, s3, s4)
    HR = NP // D

    def gath(c, b):
        return pltpu.make_async_copy(tab.at[idx_s.at[c]],
                                     rows.at[pl.ds(b * K, K)], sem_g[b])

    def scat(c, b):
        return pltpu.make_async_copy(rows.at[pl.ds(b * K, K)],
                                     acc.at[idx_d.at[c]], sem_s[b])

    pltpu.sync_copy(src.at[wid], idx_s)
    pltpu.sync_copy(dst.at[wid], idx_d)

    zeroacc = pltpu.make_async_copy(zz, acc.at[pl.ds(sid * RPT, RPT)], z0)
    zeroacc.start()
    gath(0, 0).start()
    gath(1, 1).start()
    gath(2, 2).start()

    hz = pltpu.make_async_copy(zz.at[pl.ds(0, HR)],
                               rows.at[pl.ds(3 * K, HR)], z1)
    hz.start()
    hz.wait()
    ones = jnp.ones((16,), jnp.float32)
    tailmask = lax.iota(jnp.int32, 16) >= 8

    def hadd1(v, mask=None):
        plsc.addupdate_scatter(rows, [(v >> 7) + 3 * K, v & 127], ones,
                               mask=mask)

    def hadd(c, carry):
        hadd1(idx_d[c, pl.ds(0, 16)])
        hadd1(idx_d[c, pl.ds(16, 16)])
        hadd1(idx_d[c, pl.ds(24, 16)], tailmask)
        return carry

    lax.fori_loop(0, NCHUNK, hadd, 0)
    pltpu.sync_copy(rows.at[pl.ds(3 * K, HR)], out_deg.at[wid])

    zeroacc.wait()
    plsc.subcore_barrier()

    gath(0, 0).wait()
    scat(0, 0).start(add=True)
    gath(3, 3).start()
    gath(1, 1).wait()
    scat(1, 1).start(add=True)
    gath(4, 4).start()
    for c in (2, 3, 4):
        scat(c - 2, c - 2).wait()
        gath(c + 3, c - 2).start()
        gath(c, c).wait()
        scat(c, c).start(add=True)

    def group(m, carry):
        for j in range(5):
            c = 5 * m + j
            bn = (j + 3) % 5
            scat(c - 2, bn).wait()
            nxt = jnp.minimum(c + 3, NCHUNK - 1)
            gath(nxt, bn).start()
            gath(c, j).wait()
            scat(c, j).start(add=True)
        return carry

    lax.fori_loop(1, NCHUNK // 5, group, 0)
    scat(NCHUNK - 2, 3).wait()
    scat(NCHUNK - 1, 4).wait()
    gath(NCHUNK - 1, 0).wait()
    gath(NCHUNK - 1, 1).wait()
    gath(NCHUNK - 1, 2).wait()
    plsc.subcore_barrier()

    pltpu.sync_copy(acc.at[pl.ds(sid * RPT, RPT)],
                    out.at[cid, pl.ds(sid * RPT, RPT)])


R = 1000


def _affine_body(p_ref, deg_ref, w_ref, b_ref, o_ref, *, relu):
    p = p_ref[0] + p_ref[1]
    a = p / deg_ref[...]
    h = jnp.dot(a, w_ref[...], preferred_element_type=jnp.float32) + b_ref[...]
    if relu:
        h = jnp.maximum(h, 0.0)
    o_ref[...] = h


def _tc_affine(partials, deg, w, b, *, relu):
    return pl.pallas_call(
        functools.partial(_affine_body, relu=relu),
        grid=(N // R,),
        in_specs=[
            pl.BlockSpec((NC, R, D), lambda i: (0, i, 0)),
            pl.BlockSpec((R, 1), lambda i: (i, 0)),
            pl.BlockSpec((D, D), lambda i: (0, 0)),
            pl.BlockSpec((1, D), lambda i: (0, 0)),
        ],
        out_specs=pl.BlockSpec((R, D), lambda i: (i, 0)),
        out_shape=jax.ShapeDtypeStruct((N, D), jnp.float32),
    )(partials, deg, w, b.reshape(1, D))


def kernel(x, edge_index_list, W0, b0, W1, b1):
    zz = jnp.zeros((RPT, D), jnp.float32)
    e = edge_index_list.reshape(2, 2, NW, NCHUNK, K)

    p1, hist1 = _sc_aggregate(x, e[0, 0], e[0, 1], zz)
    deg1 = jnp.maximum(hist1.sum(axis=0).reshape(NP)[:N], 1.0)[:, None]
    h1 = _tc_affine(p1, deg1, W0, b0, relu=True)
    p2, hist2 = _sc_aggregate(h1, e[1, 0], e[1, 1], zz)
    deg2 = jnp.maximum(hist2.sum(axis=0).reshape(NP)[:N], 1.0)[:, None]
    out = _tc_affine(p2, deg2, W1, b1, relu=False)
    return out


# speedup vs baseline: 12.7763x; 1.0481x over previous
import functools

import jax
import jax.numpy as jnp
from jax import lax
from jax.experimental import pallas as pl
from jax.experimental.pallas import tpu as pltpu
from jax.experimental.pallas import tpu_sc as plsc

N = 10000
D = 128
E = 320000
NC = 2
NS = 16
NW = NC * NS
EPW = E // NW
K = 40
NCHUNK = EPW // K
NP = 10240
RPT = NP // NS


@functools.partial(
    pl.kernel,
    mesh=plsc.VectorSubcoreMesh(core_axis_name="c", subcore_axis_name="s"),
    out_type=(
        jax.ShapeDtypeStruct((NC, NP, D), jnp.float32),
        jax.ShapeDtypeStruct((NW, NP // D, D), jnp.float32),
    ),
    scratch_types=[
        pltpu.VMEM_SHARED((NP, D), jnp.float32),
        pltpu.VMEM((NCHUNK, K), jnp.int32),
        pltpu.VMEM((NCHUNK, K), jnp.int32),
        pltpu.VMEM((5 * K, D), jnp.float32),
        pltpu.SemaphoreType.DMA,
        pltpu.SemaphoreType.DMA,
        pltpu.SemaphoreType.DMA,
        pltpu.SemaphoreType.DMA,
        pltpu.SemaphoreType.DMA,
        pltpu.SemaphoreType.DMA,
        pltpu.SemaphoreType.DMA,
        pltpu.SemaphoreType.DMA,
        pltpu.SemaphoreType.DMA,
        pltpu.SemaphoreType.DMA,
        pltpu.SemaphoreType.DMA,
        pltpu.SemaphoreType.DMA,
    ],
    compiler_params=pltpu.CompilerParams(use_tc_tiling_on_sc=False,
                                         needs_layout_passes=False),
)
def _sc_aggregate(tab, src, dst, zz, out, out_deg, acc, idx_s, idx_d, rows,
                  g0, g1, g2, g3, g4, s0, s1, s2, s3, s4, z0, z1):
    cid = lax.axis_index("c")
    sid = lax.axis_index("s")
    wid = sid * NC + cid
    sem_g = (g0, g1, g2, g3, g4)
    sem_s = (s0, s1, s2, s3, s4)
    HR = NP // D

    def gath(c, b):
        return pltpu.make_async_copy(tab.at[idx_s.at[c]],
                                     rows.at[pl.ds(b * K, K)], sem_g[b])

    def scat(c, b):
        return pltpu.make_async_copy(rows.at[pl.ds(b * K, K)],
                                     acc.at[idx_d.at[c]], sem_s[b])

    pltpu.sync_copy(src.at[wid], idx_s)
    pltpu.sync_copy(dst.at[wid], idx_d)

    zeroacc = pltpu.make_async_copy(zz, acc.at[pl.ds(sid * RPT, RPT)], z0)
    zeroacc.start()
    gath(0, 0).start()
    gath(1, 1).start()
    gath(2, 2).start()

    hz = pltpu.make_async_copy(zz.at[pl.ds(0, HR)],
                               rows.at[pl.ds(3 * K, HR)], z1)
    hz.start()
    hz.wait()
    ones = jnp.ones((16,), jnp.float32)
    tailmask = lax.iota(jnp.int32, 16) >= 8

    def hadd1(v, mask=None):
        plsc.addupdate_scatter(rows, [(v >> 7) + 3 * K, v & 127], ones,
                               mask=mask)

    def hadd(c, carry):
        hadd1(idx_d[c, pl.ds(0, 16)])
        hadd1(idx_d[c, pl.ds(16, 16)])
        hadd1(idx_d[c, pl.ds(24, 16)], tailmask)
        return carry

    lax.fori_loop(0, NCHUNK, hadd, 0)
    pltpu.sync_copy(rows.at[pl.ds(3 * K, HR)], out_deg.at[wid])

    zeroacc.wait()
    plsc.subcore_barrier()

    gath(0, 0).wait()
    scat(0, 0).start(add=True)
    gath(3, 3).start()
    gath(1, 1).wait()
    scat(1, 1).start(add=True)
    gath(4, 4).start()
    for c in (2, 3, 4):
        scat(c - 2, c - 2).wait()
        gath(c + 3, c - 2).start()
        gath(c, c).wait()
        scat(c, c).start(add=True)

    def group(m, carry):
        for j in range(5):
            c = 5 * m + j
            bn = (j + 3) % 5
            scat(c - 2, bn).wait()
            nxt = jnp.minimum(c + 3, NCHUNK - 1)
            gath(nxt, bn).start()
            gath(c, j).wait()
            scat(c, j).start(add=True)
        return carry

    lax.fori_loop(1, NCHUNK // 5, group, 0)
    scat(NCHUNK - 2, 3).wait()
    scat(NCHUNK - 1, 4).wait()
    gath(NCHUNK - 1, 0).wait()
    gath(NCHUNK - 1, 1).wait()
    gath(NCHUNK - 1, 2).wait()
    plsc.subcore_barrier()

    pltpu.sync_copy(acc.at[pl.ds(sid * RPT, RPT)],
                    out.at[cid, pl.ds(sid * RPT, RPT)])


R = 1024
HB = R // D


def _affine_body(p_ref, h_ref, w_ref, b_ref, o_ref, *, relu):
    p = p_ref[0] + p_ref[1]
    deg8 = jnp.maximum(h_ref[...].sum(axis=0), 1.0)
    ri = lax.broadcasted_iota(jnp.int32, (R, HB), 0)
    si = lax.broadcasted_iota(jnp.int32, (R, HB), 1)
    sel = ((ri >> 7) == si).astype(jnp.float32)
    brows = jnp.dot(sel, deg8, preferred_element_type=jnp.float32)
    ii = lax.broadcasted_iota(jnp.int32, (R, D), 0)
    jj = lax.broadcasted_iota(jnp.int32, (R, D), 1)
    deg = jnp.sum(jnp.where((ii & 127) == jj, brows, 0.0), axis=1,
                  keepdims=True)
    a = p / deg
    h = jnp.dot(a, w_ref[...], preferred_element_type=jnp.float32) + b_ref[...]
    if relu:
        h = jnp.maximum(h, 0.0)
    o_ref[...] = h


def _tc_affine(partials, hist, w, b, *, relu):
    return pl.pallas_call(
        functools.partial(_affine_body, relu=relu),
        grid=(NP // R,),
        in_specs=[
            pl.BlockSpec((NC, R, D), lambda i: (0, i, 0)),
            pl.BlockSpec((NW, HB, D), lambda i: (0, i, 0)),
            pl.BlockSpec((D, D), lambda i: (0, 0)),
            pl.BlockSpec((1, D), lambda i: (0, 0)),
        ],
        out_specs=pl.BlockSpec((R, D), lambda i: (i, 0)),
        out_shape=jax.ShapeDtypeStruct((N, D), jnp.float32),
    )(partials, hist, w, b.reshape(1, D))


def kernel(x, edge_index_list, W0, b0, W1, b1):
    zz = jnp.zeros((RPT, D), jnp.float32)
    e = edge_index_list.reshape(2, 2, NW, NCHUNK, K)

    p1, hist1 = _sc_aggregate(x, e[0, 0], e[0, 1], zz)
    h1 = _tc_affine(p1, hist1, W0, b0, relu=True)
    p2, hist2 = _sc_aggregate(h1, e[1, 0], e[1, 1], zz)
    out = _tc_affine(p2, hist2, W1, b1, relu=False)
    return out
